# baseline (speedup 1.0000x reference)
import functools

import jax
import jax.numpy as jnp
from jax import lax
from jax.experimental import pallas as pl
from jax.experimental.pallas import tpu as pltpu
from jax.experimental.pallas import tpu_sc as plsc

_NC = 2
_NS = 16
_NW = _NC * _NS
_DEGW = 16
_PADN = 16


def _make_seg_sum(N, E, D, with_deg):
  C = 128
  NCHUNK = -(-E // (_NW * C))
  NCHUNK += NCHUNK % 2
  EP = _NW * NCHUNK * C
  NP = N + _PADN
  ACCR = -(-NP // (_NS * 8)) * _NS * 8
  RZ = ACCR // _NS
  R0 = N // _NS // 8 * 8 * _NS
  RPT = R0 // _NS
  RLAST = N - 15 * RPT
  assert RZ % 8 == 0 and RPT % 8 == 0 and 15 * RPT + RLAST == N

  out_type = [jax.ShapeDtypeStruct((_NC, N, D), jnp.float32)]
  scratch = [
      pltpu.VMEM((NCHUNK, C), jnp.int32),
      pltpu.VMEM((NCHUNK, C), jnp.int32),
      pltpu.VMEM((C, D), jnp.float32),
      pltpu.VMEM((C, D), jnp.float32),
      pltpu.VMEM((RZ, D), jnp.float32),
      pltpu.VMEM_SHARED((ACCR, D), jnp.float32),
      pltpu.SemaphoreType.DMA,
      pltpu.SemaphoreType.DMA,
      pltpu.SemaphoreType.DMA,
      pltpu.SemaphoreType.DMA,
  ]
  if with_deg:
    out_type.append(jax.ShapeDtypeStruct((_NC, N, _DEGW), jnp.float32))
    scratch += [
        pltpu.VMEM((C, _DEGW), jnp.float32),
        pltpu.VMEM((RZ, _DEGW), jnp.float32),
        pltpu.VMEM_SHARED((ACCR, _DEGW), jnp.float32),
        pltpu.SemaphoreType.DMA,
    ]

  mesh = plsc.VectorSubcoreMesh(core_axis_name="c", subcore_axis_name="s")

  @functools.partial(
      pl.kernel, out_type=tuple(out_type), mesh=mesh,
      scratch_types=tuple(scratch),
      compiler_params=pltpu.CompilerParams(use_tc_tiling_on_sc=False))
  def seg(y_hbm, src_hbm, dst_hbm, *refs):
    if with_deg:
      (agg_out, deg_out, srcv, dstv, rows_a, rows_b, zb, acc, sga, sgb,
       ssa, ssb, ones, zd, dacc, sd) = refs
    else:
      (agg_out, srcv, dstv, rows_a, rows_b, zb, acc, sga, sgb, ssa,
       ssb) = refs
    c = lax.axis_index("c")
    s = lax.axis_index("s")
    wid = c * _NS + s

    pltpu.sync_copy(src_hbm.at[pl.ds(wid * NCHUNK, NCHUNK)], srcv)
    pltpu.sync_copy(dst_hbm.at[pl.ds(wid * NCHUNK, NCHUNK)], dstv)

    def zrow(j, _):
      for t in range(D // 16):
        zb[j, pl.ds(16 * t, 16)] = jnp.zeros((16,), jnp.float32)
      return 0
    lax.fori_loop(0, RZ, zrow, 0)
    if with_deg:
      def onesrow(j, _):
        for t in range(_DEGW // 16):
          ones[j, pl.ds(16 * t, 16)] = jnp.full((16,), 1.0, jnp.float32)
        return 0
      lax.fori_loop(0, C, onesrow, 0)
      def zdrow(j, _):
        for t in range(_DEGW // 16):
          zd[j, pl.ds(16 * t, 16)] = jnp.zeros((16,), jnp.float32)
        return 0
      lax.fori_loop(0, RZ, zdrow, 0)

    pltpu.sync_copy(zb, acc.at[pl.ds(s * RZ, RZ)])
    if with_deg:
      pltpu.sync_copy(zd, dacc.at[pl.ds(s * RZ, RZ)])
    plsc.subcore_barrier()

    def fg(ch, buf, sm):
      pltpu.async_copy(y_hbm.at[srcv.at[ch]], buf, sm)

    def dg(buf, sm):
      pltpu.make_async_copy(y_hbm.at[srcv.at[0]], buf, sm).wait()

    def fs(ch, buf, sm):
      pltpu.async_copy(buf, acc.at[dstv.at[ch]], sm, add=True)

    def ds(buf, sm):
      pltpu.make_async_copy(buf, acc.at[dstv.at[0]], sm).wait()

    def fd(ch):
      pltpu.async_copy(ones, dacc.at[dstv.at[ch]], sd, add=True)

    def dd():
      pltpu.make_async_copy(ones, dacc.at[dstv.at[0]], sd).wait()

    assert NCHUNK % 2 == 0
    fg(0, rows_a, sga)
    fg(1, rows_b, sgb)

    def step(i, _):
      c0 = 2 * i
      dg(rows_a, sga)
      fs(c0, rows_a, ssa)
      dg(rows_b, sgb)
      fs(c0 + 1, rows_b, ssb)
      if with_deg:
        fd(c0)
        fd(c0 + 1)
      ds(rows_a, ssa)
      fg(c0 + 2, rows_a, sga)
      ds(rows_b, ssb)
      fg(c0 + 3, rows_b, sgb)
      if with_deg:
        dd()
        dd()
      return 0
    lax.fori_loop(0, NCHUNK // 2 - 1, step, 0)

    dg(rows_a, sga)
    fs(NCHUNK - 2, rows_a, ssa)
    dg(rows_b, sgb)
    fs(NCHUNK - 1, rows_b, ssb)
    if with_deg:
      fd(NCHUNK - 2)
      fd(NCHUNK - 1)
      dd()
      dd()
    ds(rows_a, ssa)
    ds(rows_b, ssb)
    plsc.subcore_barrier()

    @pl.when(s < _NS - 1)
    def _():
      r = s * RPT
      pltpu.sync_copy(acc.at[pl.ds(r, RPT)], agg_out.at[c, pl.ds(r, RPT)])
      if with_deg:
        pltpu.sync_copy(dacc.at[pl.ds(r, RPT)], deg_out.at[c, pl.ds(r, RPT)])

    @pl.when(s == _NS - 1)
    def _():
      r = (_NS - 1) * RPT
      pltpu.sync_copy(acc.at[pl.ds(r, RLAST)],
                      agg_out.at[c, pl.ds(r, RLAST)])
      if with_deg:
        pltpu.sync_copy(dacc.at[pl.ds(r, RLAST)],
                        deg_out.at[c, pl.ds(r, RLAST)])

  def run(y, src, dst):
    pad = EP - E
    srcp = jnp.concatenate(
        [src, jnp.full((pad,), N, jnp.int32)]).reshape(EP // C, C)
    dstp = jnp.concatenate(
        [dst, jnp.full((pad,), N, jnp.int32)]).reshape(EP // C, C)
    return seg(y, srcp, dstp)

  return run


def _mm_body(x_ref, w_ref, o_ref):
  res = jnp.dot(x_ref[...], w_ref[...], preferred_element_type=jnp.float32)
  o_ref[...] = jnp.concatenate(
      [res, jnp.zeros((_PADN, res.shape[1]), jnp.float32)], axis=0)


def _mid_body(aggp_ref, degp_ref, b1_ref, w2_ref, y2_ref, degi_ref):
  agg = aggp_ref[0] + aggp_ref[1]
  deg = degp_ref[0] + degp_ref[1]
  degi = 1.0 / jnp.maximum(deg, 1.0)
  h = agg * jnp.concatenate([degi, degi], axis=1) + b1_ref[...]
  h = jnp.maximum(h, 0.0)
  y2 = jnp.dot(h, w2_ref[...], preferred_element_type=jnp.float32)
  y2_ref[...] = jnp.concatenate(
      [y2, jnp.zeros((_PADN, y2.shape[1]), jnp.float32)], axis=0)
  degi_ref[...] = degi


def _fin_body(aggp_ref, degi_ref, b2_ref, o_ref):
  agg = aggp_ref[0] + aggp_ref[1]
  res = jnp.maximum(agg * degi_ref[...] + b2_ref[...], 0.0)
  o_ref[...] = res[:, :o_ref.shape[1]]


def kernel(features, edge_index, W1, b1, W2, b2):
  N, _ = features.shape
  E = edge_index.shape[1]
  D_HID = W1.shape[1]
  D_OUT = W2.shape[1]
  src = edge_index[0]
  dst = edge_index[1]

  y1 = pl.pallas_call(
      _mm_body,
      out_shape=jax.ShapeDtypeStruct((N + _PADN, D_HID), jnp.float32),
  )(features, W1)

  aggp, degp = _make_seg_sum(N, E, D_HID, True)(y1, src, dst)

  W2p = jnp.zeros((D_HID, 16), jnp.float32).at[:, :D_OUT].set(W2)
  y2, degi = pl.pallas_call(
      _mid_body,
      out_shape=(jax.ShapeDtypeStruct((N + _PADN, 16), jnp.float32),
                 jax.ShapeDtypeStruct((N, 16), jnp.float32)),
  )(aggp, degp, b1.reshape(1, D_HID), W2p)

  agg2p = _make_seg_sum(N, E, 16, False)(y2, src, dst)
  if isinstance(agg2p, (tuple, list)):
    agg2p = agg2p[0]

  b2p = jnp.zeros((1, 16), jnp.float32).at[0, :D_OUT].set(b2)
  out = pl.pallas_call(
      _fin_body,
      out_shape=jax.ShapeDtypeStruct((N, D_OUT), jnp.float32),
  )(agg2p, degi, b2p)
  return out
